# baseline (speedup 1.0000x reference)
import jax
import jax.numpy as jnp
from jax import lax
from jax.experimental import pallas as pl
from jax.experimental.pallas import tpu as pltpu
from jax.experimental.pallas import tpu_sc as plsc

N, E, D, DE, DG, H = 10000, 160000, 256, 16, 128, 512

NC, NS = 2, 16
NW = NC * NS
EP = 163840
EPAD = EP - E

G_PER_W = EP // NW
G_IDX = 128
G_NCH = G_PER_W // G_IDX
G_PAIRS = G_NCH // 2

DH = D // NC
S_PER_T = EP // NS
S_IDX = 128
S_NCH = S_PER_T // S_IDX
S_PAIRS = S_NCH // 2
TRASH = 8
ACC_R = N + TRASH
ZR = 632
ZR_LAST = ACC_R - 15 * ZR
WR = 624
WR_TAIL = N - NS * WR

RB_E = 2048
RB_N = 1000


def _sc_gather_body(idx_hbm, table_hbm, out_hbm, idx_v, buf0, buf1, sem):
    c = lax.axis_index("c")
    s = lax.axis_index("s")
    w = s * NC + c
    base = w * G_PER_W
    pltpu.sync_copy(idx_hbm.at[w], idx_v)

    pltpu.async_copy(table_hbm.at[idx_v.at[0]], buf0, sem)

    def pair(p, _):
        i0 = 2 * p
        pltpu.make_async_copy(table_hbm.at[idx_v.at[i0]], buf0, sem).wait()
        pltpu.async_copy(table_hbm.at[idx_v.at[i0 + 1]], buf1, sem)
        pltpu.sync_copy(buf0, out_hbm.at[pl.ds(base + i0 * G_IDX, G_IDX), :])
        pltpu.make_async_copy(table_hbm.at[idx_v.at[i0 + 1]], buf1, sem).wait()

        @pl.when(p < G_PAIRS - 1)
        def _prefetch():
            pltpu.async_copy(table_hbm.at[idx_v.at[i0 + 2]], buf0, sem)

        pltpu.sync_copy(buf1, out_hbm.at[pl.ds(base + (i0 + 1) * G_IDX, G_IDX), :])
        return 0

    lax.fori_loop(0, G_PAIRS, pair, 0)


def _sc_scatter_body(ridx_hbm, e_hbm, out_hbm, idx_v, buf0, buf1, acc, sem):
    c = lax.axis_index("c")
    s = lax.axis_index("s")
    zero16 = jnp.zeros((16,), jnp.float32)

    def zrow(r, _):
        for k in range(DH // 16):
            buf0[r, pl.ds(k * 16, 16)] = zero16
        return 0

    lax.fori_loop(0, S_IDX, zrow, 0)

    @pl.when(s < NS - 1)
    def _zero_main():
        zb = s * ZR
        for t in range(ZR // S_IDX):
            pltpu.sync_copy(buf0, acc.at[pl.ds(zb + t * S_IDX, S_IDX), :])
        zrem = ZR % S_IDX
        pltpu.sync_copy(buf0.at[pl.ds(0, zrem), :],
                        acc.at[pl.ds(zb + ZR - zrem, zrem), :])

    @pl.when(s == NS - 1)
    def _zero_last():
        zb = (NS - 1) * ZR
        for t in range(ZR_LAST // S_IDX):
            pltpu.sync_copy(buf0, acc.at[pl.ds(zb + t * S_IDX, S_IDX), :])
        zrem = ZR_LAST % S_IDX
        pltpu.sync_copy(buf0.at[pl.ds(0, zrem), :],
                        acc.at[pl.ds(zb + ZR_LAST - zrem, zrem), :])

    pltpu.sync_copy(ridx_hbm.at[s], idx_v)
    plsc.subcore_barrier()

    row0 = s * S_PER_T
    col = c * DH
    pltpu.async_copy(e_hbm.at[pl.ds(row0, S_IDX), pl.ds(col, DH)], buf0, sem)

    def pair(p, _):
        i0 = 2 * p
        pltpu.make_async_copy(e_hbm.at[pl.ds(row0 + i0 * S_IDX, S_IDX),
                                       pl.ds(col, DH)], buf0, sem).wait()
        pltpu.async_copy(e_hbm.at[pl.ds(row0 + (i0 + 1) * S_IDX, S_IDX),
                                  pl.ds(col, DH)], buf1, sem)
        pltpu.sync_copy(buf0, acc.at[idx_v.at[i0]], add=True)
        pltpu.make_async_copy(e_hbm.at[pl.ds(row0 + (i0 + 1) * S_IDX, S_IDX),
                                       pl.ds(col, DH)], buf1, sem).wait()

        @pl.when(p < S_PAIRS - 1)
        def _prefetch():
            pltpu.async_copy(e_hbm.at[pl.ds(row0 + (i0 + 2) * S_IDX, S_IDX),
                                      pl.ds(col, DH)], buf0, sem)

        pltpu.sync_copy(buf1, acc.at[idx_v.at[i0 + 1]], add=True)
        return 0

    lax.fori_loop(0, S_PAIRS, pair, 0)
    plsc.subcore_barrier()

    pltpu.sync_copy(acc.at[pl.ds(s * WR, WR), :],
                    out_hbm.at[pl.ds(s * WR, WR), pl.ds(col, DH)])

    @pl.when(s == NS - 1)
    def _write_tail():
        pltpu.sync_copy(acc.at[pl.ds(NS * WR, WR_TAIL), :],
                        out_hbm.at[pl.ds(NS * WR, WR_TAIL), pl.ds(col, DH)])


_gather_call = pl.kernel(
    _sc_gather_body,
    out_type=jax.ShapeDtypeStruct((EP, D // 2), jnp.int32),
    mesh=plsc.VectorSubcoreMesh(core_axis_name="c", subcore_axis_name="s"),
    scratch_types=[
        pltpu.VMEM((G_NCH, G_IDX), jnp.int32),
        pltpu.VMEM((G_IDX, D // 2), jnp.int32),
        pltpu.VMEM((G_IDX, D // 2), jnp.int32),
        pltpu.SemaphoreType.DMA,
    ],
)

_scatter_call = pl.kernel(
    _sc_scatter_body,
    out_type=jax.ShapeDtypeStruct((N, D), jnp.float32),
    mesh=plsc.VectorSubcoreMesh(core_axis_name="c", subcore_axis_name="s"),
    scratch_types=[
        pltpu.VMEM((S_NCH, S_IDX), jnp.int32),
        pltpu.VMEM((S_IDX, DH), jnp.float32),
        pltpu.VMEM((S_IDX, DH), jnp.float32),
        pltpu.VMEM_SHARED((ACC_R, DH), jnp.float32),
        pltpu.SemaphoreType.DMA,
    ],
)


def _edge_tc(sent_ref, edges_ref, we_ref, be_ref, out_ref):
    z = jnp.dot(edges_ref[...], we_ref[...], preferred_element_type=jnp.float32)
    x = sent_ref[...].astype(jnp.float32) + z + be_ref[...]
    u = jnp.exp(jnp.minimum(x, 30.0))
    u = u * (u + 2.0)
    out_ref[...] = x * u / (u + 2.0)


def _mlp_tc(nodes_ref, recv_ref, g_ref, eps_ref, w1a_ref, w1b_ref, b1_ref,
            w2_ref, b2_ref, out_ref):
    h = (1.0 + eps_ref[...]) * nodes_ref[...] + recv_ref[...]
    gv = jnp.dot(g_ref[...], w1b_ref[...], preferred_element_type=jnp.float32) + b1_ref[...]
    t = jnp.maximum(jnp.dot(h, w1a_ref[...], preferred_element_type=jnp.float32) + gv, 0.0)
    out_ref[...] = jnp.dot(t, w2_ref[...], preferred_element_type=jnp.float32) + b2_ref[...]


def kernel(nodes, edges, globals_, senders, receivers, epsilon,
           W_e_kernel, W_e_bias, W1, b1, W2, b2):
    senders_p = jnp.concatenate(
        [senders, jnp.zeros((EPAD,), jnp.int32)]).reshape(NW, G_NCH, G_IDX)
    nodes_packed = jax.lax.bitcast_convert_type(
        nodes.astype(jnp.bfloat16).reshape(N, D // 2, 2), jnp.int32)
    sent = jax.lax.bitcast_convert_type(
        _gather_call(senders_p, nodes_packed), jnp.bfloat16).reshape(EP, D)

    edges_p = jnp.concatenate([edges, jnp.zeros((EPAD, DE), jnp.float32)])
    e = pl.pallas_call(
        _edge_tc,
        grid=(EP // RB_E,),
        in_specs=[
            pl.BlockSpec((RB_E, D), lambda i: (i, 0)),
            pl.BlockSpec((RB_E, DE), lambda i: (i, 0)),
            pl.BlockSpec((DE, D), lambda i: (0, 0)),
            pl.BlockSpec((1, D), lambda i: (0, 0)),
        ],
        out_specs=pl.BlockSpec((RB_E, D), lambda i: (i, 0)),
        out_shape=jax.ShapeDtypeStruct((EP, D), jnp.float32),
    )(sent, edges_p, W_e_kernel, W_e_bias.reshape(1, D))

    pad_ids = N + (jnp.arange(EPAD, dtype=jnp.int32) % TRASH)
    receivers_p = jnp.concatenate(
        [receivers, pad_ids]).reshape(NS, S_NCH, S_IDX)
    recv = _scatter_call(receivers_p, e)

    out = pl.pallas_call(
        _mlp_tc,
        grid=(N // RB_N,),
        in_specs=[
            pl.BlockSpec((RB_N, D), lambda i: (i, 0)),
            pl.BlockSpec((RB_N, D), lambda i: (i, 0)),
            pl.BlockSpec((1, DG), lambda i: (0, 0)),
            pl.BlockSpec((1, 1), lambda i: (0, 0)),
            pl.BlockSpec((D, H), lambda i: (0, 0)),
            pl.BlockSpec((DG, H), lambda i: (0, 0)),
            pl.BlockSpec((1, H), lambda i: (0, 0)),
            pl.BlockSpec((H, D), lambda i: (0, 0)),
            pl.BlockSpec((1, D), lambda i: (0, 0)),
        ],
        out_specs=pl.BlockSpec((RB_N, D), lambda i: (i, 0)),
        out_shape=jax.ShapeDtypeStruct((N, D), jnp.float32),
    )(nodes, recv, globals_, epsilon, W1[:D], W1[D:], b1.reshape(1, H),
      W2, b2.reshape(1, D))
    return out


# speedup vs baseline: 2.1957x; 2.1957x over previous
import jax
import jax.numpy as jnp
from jax import lax
from jax.experimental import pallas as pl
from jax.experimental.pallas import tpu as pltpu
from jax.experimental.pallas import tpu_sc as plsc

N, E, D, DE, DG, H = 10000, 160000, 256, 16, 128, 512

NC, NS = 2, 16
NW = NC * NS
EP = 163840
EPAD = EP - E

G_PER_W = EP // NW
G_IDX = 128
G_NCH = G_PER_W // G_IDX
G_PAIRS = G_NCH // 2

DH = D // NC
S_PER_T = EP // NS
S_IDX = 128
S_NCH = S_PER_T // S_IDX
S_PAIRS = S_NCH // 2
TRASH = 8
ACC_R = N + TRASH
ZR = 632
ZR_LAST = ACC_R - 15 * ZR
WR = 624
WR_TAIL = N - NS * WR

RB_E = 2048
RB_N = 1000


def _sc_gather_body(idx_hbm, table_hbm, out_hbm, idx_v, buf0, buf1, sem):
    c = lax.axis_index("c")
    s = lax.axis_index("s")
    w = s * NC + c
    base = w * G_PER_W
    pltpu.sync_copy(idx_hbm.at[w], idx_v)

    pltpu.async_copy(table_hbm.at[idx_v.at[0]], buf0, sem)

    def pair(p, _):
        i0 = 2 * p
        pltpu.make_async_copy(table_hbm.at[idx_v.at[i0]], buf0, sem).wait()
        pltpu.async_copy(table_hbm.at[idx_v.at[i0 + 1]], buf1, sem)
        pltpu.sync_copy(buf0, out_hbm.at[pl.ds(base + i0 * G_IDX, G_IDX), :])
        pltpu.make_async_copy(table_hbm.at[idx_v.at[i0 + 1]], buf1, sem).wait()

        @pl.when(p < G_PAIRS - 1)
        def _prefetch():
            pltpu.async_copy(table_hbm.at[idx_v.at[i0 + 2]], buf0, sem)

        pltpu.sync_copy(buf1, out_hbm.at[pl.ds(base + (i0 + 1) * G_IDX, G_IDX), :])
        return 0

    lax.fori_loop(0, G_PAIRS, pair, 0)


def _sc_scatter_body(ridx_hbm, e_hbm, out_hbm, idx_v, buf0, buf1, acc, sem):
    c = lax.axis_index("c")
    s = lax.axis_index("s")
    zero16 = jnp.zeros((16,), jnp.float32)

    def zrow(r, _):
        for k in range(DH // 16):
            buf0[r, pl.ds(k * 16, 16)] = zero16
        return 0

    lax.fori_loop(0, S_IDX, zrow, 0)

    @pl.when(s < NS - 1)
    def _zero_main():
        zb = s * ZR
        for t in range(ZR // S_IDX):
            pltpu.sync_copy(buf0, acc.at[pl.ds(zb + t * S_IDX, S_IDX), :])
        zrem = ZR % S_IDX
        pltpu.sync_copy(buf0.at[pl.ds(0, zrem), :],
                        acc.at[pl.ds(zb + ZR - zrem, zrem), :])

    @pl.when(s == NS - 1)
    def _zero_last():
        zb = (NS - 1) * ZR
        for t in range(ZR_LAST // S_IDX):
            pltpu.sync_copy(buf0, acc.at[pl.ds(zb + t * S_IDX, S_IDX), :])
        zrem = ZR_LAST % S_IDX
        pltpu.sync_copy(buf0.at[pl.ds(0, zrem), :],
                        acc.at[pl.ds(zb + ZR_LAST - zrem, zrem), :])

    pltpu.sync_copy(ridx_hbm.at[s], idx_v)
    plsc.subcore_barrier()

    row0 = s * S_PER_T
    col = c * DH
    pltpu.async_copy(e_hbm.at[pl.ds(row0, S_IDX), pl.ds(col, DH)], buf0, sem)

    def pair(p, _):
        i0 = 2 * p
        pltpu.make_async_copy(e_hbm.at[pl.ds(row0 + i0 * S_IDX, S_IDX),
                                       pl.ds(col, DH)], buf0, sem).wait()
        pltpu.async_copy(e_hbm.at[pl.ds(row0 + (i0 + 1) * S_IDX, S_IDX),
                                  pl.ds(col, DH)], buf1, sem)
        pltpu.sync_copy(buf0, acc.at[idx_v.at[i0]], add=True)
        pltpu.make_async_copy(e_hbm.at[pl.ds(row0 + (i0 + 1) * S_IDX, S_IDX),
                                       pl.ds(col, DH)], buf1, sem).wait()

        @pl.when(p < S_PAIRS - 1)
        def _prefetch():
            pltpu.async_copy(e_hbm.at[pl.ds(row0 + (i0 + 2) * S_IDX, S_IDX),
                                      pl.ds(col, DH)], buf0, sem)

        pltpu.sync_copy(buf1, acc.at[idx_v.at[i0 + 1]], add=True)
        return 0

    lax.fori_loop(0, S_PAIRS, pair, 0)
    plsc.subcore_barrier()

    pltpu.sync_copy(acc.at[pl.ds(s * WR, WR), :],
                    out_hbm.at[pl.ds(s * WR, WR), pl.ds(col, DH)])

    @pl.when(s == NS - 1)
    def _write_tail():
        pltpu.sync_copy(acc.at[pl.ds(NS * WR, WR_TAIL), :],
                        out_hbm.at[pl.ds(NS * WR, WR_TAIL), pl.ds(col, DH)])


_gather_call = pl.kernel(
    _sc_gather_body,
    out_type=jax.ShapeDtypeStruct((EP, D // 2), jnp.int32),
    mesh=plsc.VectorSubcoreMesh(core_axis_name="c", subcore_axis_name="s"),
    scratch_types=[
        pltpu.VMEM((G_NCH, G_IDX), jnp.int32),
        pltpu.VMEM((G_IDX, D // 2), jnp.int32),
        pltpu.VMEM((G_IDX, D // 2), jnp.int32),
        pltpu.SemaphoreType.DMA,
    ],
)

_scatter_call = pl.kernel(
    _sc_scatter_body,
    out_type=jax.ShapeDtypeStruct((N, D), jnp.float32),
    mesh=plsc.VectorSubcoreMesh(core_axis_name="c", subcore_axis_name="s"),
    scratch_types=[
        pltpu.VMEM((S_NCH, S_IDX), jnp.int32),
        pltpu.VMEM((S_IDX, DH), jnp.float32),
        pltpu.VMEM((S_IDX, DH), jnp.float32),
        pltpu.VMEM_SHARED((ACC_R, DH), jnp.float32),
        pltpu.SemaphoreType.DMA,
    ],
)


def _edge_tc(sent_ref, edges_ref, we_ref, be_ref, out_ref):
    z = jnp.dot(edges_ref[...], we_ref[...], preferred_element_type=jnp.float32)
    packed = sent_ref[...]
    lo = jax.lax.bitcast_convert_type(packed << 16, jnp.float32)
    hi = jax.lax.bitcast_convert_type(packed & jnp.int32(-65536), jnp.float32)
    sent = jnp.concatenate([lo, hi], axis=1)
    x = sent + z + be_ref[...]
    u = jnp.exp(jnp.minimum(x, 30.0))
    u = u * (u + 2.0)
    out_ref[...] = x * u / (u + 2.0)


def _mlp_tc(nodes_ref, recv_ref, g_ref, eps_ref, w1a_ref, w1b_ref, b1_ref,
            w2_ref, b2_ref, out_ref):
    h = (1.0 + eps_ref[...]) * nodes_ref[...] + recv_ref[...]
    gv = jnp.dot(g_ref[...], w1b_ref[...], preferred_element_type=jnp.float32) + b1_ref[...]
    t = jnp.maximum(jnp.dot(h, w1a_ref[...], preferred_element_type=jnp.float32) + gv, 0.0)
    out_ref[...] = jnp.dot(t, w2_ref[...], preferred_element_type=jnp.float32) + b2_ref[...]


def kernel(nodes, edges, globals_, senders, receivers, epsilon,
           W_e_kernel, W_e_bias, W1, b1, W2, b2):
    senders_p = jnp.concatenate(
        [senders, jnp.zeros((EPAD,), jnp.int32)]).reshape(NW, G_NCH, G_IDX)
    lo16 = jax.lax.bitcast_convert_type(
        nodes[:, :D // 2].astype(jnp.bfloat16), jnp.uint16).astype(jnp.uint32)
    hi16 = jax.lax.bitcast_convert_type(
        nodes[:, D // 2:].astype(jnp.bfloat16), jnp.uint16).astype(jnp.uint32)
    nodes_packed = ((hi16 << 16) | lo16).astype(jnp.int32)
    sent = _gather_call(senders_p, nodes_packed)

    edges_p = jnp.concatenate([edges, jnp.zeros((EPAD, DE), jnp.float32)])
    e = pl.pallas_call(
        _edge_tc,
        grid=(EP // RB_E,),
        in_specs=[
            pl.BlockSpec((RB_E, D // 2), lambda i: (i, 0)),
            pl.BlockSpec((RB_E, DE), lambda i: (i, 0)),
            pl.BlockSpec((DE, D), lambda i: (0, 0)),
            pl.BlockSpec((1, D), lambda i: (0, 0)),
        ],
        out_specs=pl.BlockSpec((RB_E, D), lambda i: (i, 0)),
        out_shape=jax.ShapeDtypeStruct((EP, D), jnp.float32),
    )(sent, edges_p, W_e_kernel, W_e_bias.reshape(1, D))

    pad_ids = N + (jnp.arange(EPAD, dtype=jnp.int32) % TRASH)
    receivers_p = jnp.concatenate(
        [receivers, pad_ids]).reshape(NS, S_NCH, S_IDX)
    recv = _scatter_call(receivers_p, e)

    out = pl.pallas_call(
        _mlp_tc,
        grid=(N // RB_N,),
        in_specs=[
            pl.BlockSpec((RB_N, D), lambda i: (i, 0)),
            pl.BlockSpec((RB_N, D), lambda i: (i, 0)),
            pl.BlockSpec((1, DG), lambda i: (0, 0)),
            pl.BlockSpec((1, 1), lambda i: (0, 0)),
            pl.BlockSpec((D, H), lambda i: (0, 0)),
            pl.BlockSpec((DG, H), lambda i: (0, 0)),
            pl.BlockSpec((1, H), lambda i: (0, 0)),
            pl.BlockSpec((H, D), lambda i: (0, 0)),
            pl.BlockSpec((1, D), lambda i: (0, 0)),
        ],
        out_specs=pl.BlockSpec((RB_N, D), lambda i: (i, 0)),
        out_shape=jax.ShapeDtypeStruct((N, D), jnp.float32),
    )(nodes, recv, globals_, epsilon, W1[:D], W1[D:], b1.reshape(1, H),
      W2, b2.reshape(1, D))
    return out


# speedup vs baseline: 3.5668x; 1.6244x over previous
import jax
import jax.numpy as jnp
from jax import lax
from jax.experimental import pallas as pl
from jax.experimental.pallas import tpu as pltpu
from jax.experimental.pallas import tpu_sc as plsc

N, E, D, DE, DG, H = 10000, 160000, 256, 16, 128, 512

NC, NS = 2, 16
NW = NC * NS
EP = 163840
EPAD = EP - E

G_PER_W = EP // NW
G_IDX = 128
G_NCH = G_PER_W // G_IDX
G_PAIRS = G_NCH // 2
TLOAD = 624
TLAST = N - (NS - 1) * TLOAD

DH = D // NC
S_PER_T = EP // NS
S_IDX = 128
S_NCH = S_PER_T // S_IDX
S_PAIRS = S_NCH // 2
TRASH = 8
ACC_R = N + TRASH
ZR = 632
ZR_LAST = ACC_R - 15 * ZR
WR = 624
WR_TAIL = N - NS * WR

RB_E = 2048
RB_N = 1000


def _sc_gather_body(idx_hbm, table_hbm, out_hbm, idx_v, buf0, buf1, tbl, sem):
    c = lax.axis_index("c")
    s = lax.axis_index("s")
    w = s * NC + c
    base = w * G_PER_W

    @pl.when(s < NS - 1)
    def _load_main():
        pltpu.sync_copy(table_hbm.at[pl.ds(s * TLOAD, TLOAD), :],
                        tbl.at[pl.ds(s * TLOAD, TLOAD), :])

    @pl.when(s == NS - 1)
    def _load_last():
        pltpu.sync_copy(table_hbm.at[pl.ds((NS - 1) * TLOAD, TLAST), :],
                        tbl.at[pl.ds((NS - 1) * TLOAD, TLAST), :])

    pltpu.sync_copy(idx_hbm.at[w], idx_v)
    plsc.subcore_barrier()

    pltpu.async_copy(tbl.at[idx_v.at[0]], buf0, sem)

    def pair(p, _):
        i0 = 2 * p
        pltpu.make_async_copy(tbl.at[idx_v.at[i0]], buf0, sem).wait()
        pltpu.async_copy(tbl.at[idx_v.at[i0 + 1]], buf1, sem)
        pltpu.sync_copy(buf0, out_hbm.at[pl.ds(base + i0 * G_IDX, G_IDX), :])
        pltpu.make_async_copy(tbl.at[idx_v.at[i0 + 1]], buf1, sem).wait()

        @pl.when(p < G_PAIRS - 1)
        def _prefetch():
            pltpu.async_copy(tbl.at[idx_v.at[i0 + 2]], buf0, sem)

        pltpu.sync_copy(buf1, out_hbm.at[pl.ds(base + (i0 + 1) * G_IDX, G_IDX), :])
        return 0

    lax.fori_loop(0, G_PAIRS, pair, 0)


def _sc_scatter_body(ridx_hbm, e_hbm, out_hbm, idx_v, buf0, buf1, acc, sem):
    c = lax.axis_index("c")
    s = lax.axis_index("s")
    zero16 = jnp.zeros((16,), jnp.float32)

    def zrow(r, _):
        for k in range(DH // 16):
            buf0[r, pl.ds(k * 16, 16)] = zero16
        return 0

    lax.fori_loop(0, S_IDX, zrow, 0)

    @pl.when(s < NS - 1)
    def _zero_main():
        zb = s * ZR
        for t in range(ZR // S_IDX):
            pltpu.sync_copy(buf0, acc.at[pl.ds(zb + t * S_IDX, S_IDX), :])
        zrem = ZR % S_IDX
        pltpu.sync_copy(buf0.at[pl.ds(0, zrem), :],
                        acc.at[pl.ds(zb + ZR - zrem, zrem), :])

    @pl.when(s == NS - 1)
    def _zero_last():
        zb = (NS - 1) * ZR
        for t in range(ZR_LAST // S_IDX):
            pltpu.sync_copy(buf0, acc.at[pl.ds(zb + t * S_IDX, S_IDX), :])
        zrem = ZR_LAST % S_IDX
        pltpu.sync_copy(buf0.at[pl.ds(0, zrem), :],
                        acc.at[pl.ds(zb + ZR_LAST - zrem, zrem), :])

    pltpu.sync_copy(ridx_hbm.at[s], idx_v)
    plsc.subcore_barrier()

    row0 = s * S_PER_T
    col = c * DH
    pltpu.async_copy(e_hbm.at[pl.ds(row0, S_IDX), pl.ds(col, DH)], buf0, sem)

    def pair(p, _):
        i0 = 2 * p
        pltpu.make_async_copy(e_hbm.at[pl.ds(row0 + i0 * S_IDX, S_IDX),
                                       pl.ds(col, DH)], buf0, sem).wait()
        pltpu.async_copy(e_hbm.at[pl.ds(row0 + (i0 + 1) * S_IDX, S_IDX),
                                  pl.ds(col, DH)], buf1, sem)
        pltpu.sync_copy(buf0, acc.at[idx_v.at[i0]], add=True)
        pltpu.make_async_copy(e_hbm.at[pl.ds(row0 + (i0 + 1) * S_IDX, S_IDX),
                                       pl.ds(col, DH)], buf1, sem).wait()

        @pl.when(p < S_PAIRS - 1)
        def _prefetch():
            pltpu.async_copy(e_hbm.at[pl.ds(row0 + (i0 + 2) * S_IDX, S_IDX),
                                      pl.ds(col, DH)], buf0, sem)

        pltpu.sync_copy(buf1, acc.at[idx_v.at[i0 + 1]], add=True)
        return 0

    lax.fori_loop(0, S_PAIRS, pair, 0)
    plsc.subcore_barrier()

    pltpu.sync_copy(acc.at[pl.ds(s * WR, WR), :],
                    out_hbm.at[pl.ds(s * WR, WR), pl.ds(col, DH)])

    @pl.when(s == NS - 1)
    def _write_tail():
        pltpu.sync_copy(acc.at[pl.ds(NS * WR, WR_TAIL), :],
                        out_hbm.at[pl.ds(NS * WR, WR_TAIL), pl.ds(col, DH)])


_gather_call = pl.kernel(
    _sc_gather_body,
    out_type=jax.ShapeDtypeStruct((EP, D // 2), jnp.int32),
    mesh=plsc.VectorSubcoreMesh(core_axis_name="c", subcore_axis_name="s"),
    scratch_types=[
        pltpu.VMEM((G_NCH, G_IDX), jnp.int32),
        pltpu.VMEM((G_IDX, D // 2), jnp.int32),
        pltpu.VMEM((G_IDX, D // 2), jnp.int32),
        pltpu.VMEM_SHARED((N, D // 2), jnp.int32),
        pltpu.SemaphoreType.DMA,
    ],
)

_scatter_call = pl.kernel(
    _sc_scatter_body,
    out_type=jax.ShapeDtypeStruct((N, D), jnp.float32),
    mesh=plsc.VectorSubcoreMesh(core_axis_name="c", subcore_axis_name="s"),
    scratch_types=[
        pltpu.VMEM((S_NCH, S_IDX), jnp.int32),
        pltpu.VMEM((S_IDX, DH), jnp.float32),
        pltpu.VMEM((S_IDX, DH), jnp.float32),
        pltpu.VMEM_SHARED((ACC_R, DH), jnp.float32),
        pltpu.SemaphoreType.DMA,
    ],
)


def _edge_tc(sent_ref, edges_ref, we_ref, be_ref, out_ref):
    z = jnp.dot(edges_ref[...], we_ref[...], preferred_element_type=jnp.float32)
    packed = sent_ref[...]
    lo = jax.lax.bitcast_convert_type(packed << 16, jnp.float32)
    hi = jax.lax.bitcast_convert_type(packed & jnp.int32(-65536), jnp.float32)
    sent = jnp.concatenate([lo, hi], axis=1)
    x = sent + z + be_ref[...]
    u = jnp.exp(jnp.minimum(x, 30.0))
    u = u * (u + 2.0)
    out_ref[...] = x * u / (u + 2.0)


def _mlp_tc(nodes_ref, recv_ref, g_ref, eps_ref, w1a_ref, w1b_ref, b1_ref,
            w2_ref, b2_ref, out_ref):
    h = (1.0 + eps_ref[...]) * nodes_ref[...] + recv_ref[...]
    gv = jnp.dot(g_ref[...], w1b_ref[...], preferred_element_type=jnp.float32) + b1_ref[...]
    t = jnp.maximum(jnp.dot(h, w1a_ref[...], preferred_element_type=jnp.float32) + gv, 0.0)
    out_ref[...] = jnp.dot(t, w2_ref[...], preferred_element_type=jnp.float32) + b2_ref[...]


def kernel(nodes, edges, globals_, senders, receivers, epsilon,
           W_e_kernel, W_e_bias, W1, b1, W2, b2):
    senders_p = jnp.concatenate(
        [senders, jnp.zeros((EPAD,), jnp.int32)]).reshape(NW, G_NCH, G_IDX)
    lo16 = jax.lax.bitcast_convert_type(
        nodes[:, :D // 2].astype(jnp.bfloat16), jnp.uint16).astype(jnp.uint32)
    hi16 = jax.lax.bitcast_convert_type(
        nodes[:, D // 2:].astype(jnp.bfloat16), jnp.uint16).astype(jnp.uint32)
    nodes_packed = ((hi16 << 16) | lo16).astype(jnp.int32)
    sent = _gather_call(senders_p, nodes_packed)

    edges_p = jnp.concatenate([edges, jnp.zeros((EPAD, DE), jnp.float32)])
    e = pl.pallas_call(
        _edge_tc,
        grid=(EP // RB_E,),
        in_specs=[
            pl.BlockSpec((RB_E, D // 2), lambda i: (i, 0)),
            pl.BlockSpec((RB_E, DE), lambda i: (i, 0)),
            pl.BlockSpec((DE, D), lambda i: (0, 0)),
            pl.BlockSpec((1, D), lambda i: (0, 0)),
        ],
        out_specs=pl.BlockSpec((RB_E, D), lambda i: (i, 0)),
        out_shape=jax.ShapeDtypeStruct((EP, D), jnp.float32),
    )(sent, edges_p, W_e_kernel, W_e_bias.reshape(1, D))

    pad_ids = N + (jnp.arange(EPAD, dtype=jnp.int32) % TRASH)
    receivers_p = jnp.concatenate(
        [receivers, pad_ids]).reshape(NS, S_NCH, S_IDX)
    recv = _scatter_call(receivers_p, e)

    out = pl.pallas_call(
        _mlp_tc,
        grid=(N // RB_N,),
        in_specs=[
            pl.BlockSpec((RB_N, D), lambda i: (i, 0)),
            pl.BlockSpec((RB_N, D), lambda i: (i, 0)),
            pl.BlockSpec((1, DG), lambda i: (0, 0)),
            pl.BlockSpec((1, 1), lambda i: (0, 0)),
            pl.BlockSpec((D, H), lambda i: (0, 0)),
            pl.BlockSpec((DG, H), lambda i: (0, 0)),
            pl.BlockSpec((1, H), lambda i: (0, 0)),
            pl.BlockSpec((H, D), lambda i: (0, 0)),
            pl.BlockSpec((1, D), lambda i: (0, 0)),
        ],
        out_specs=pl.BlockSpec((RB_N, D), lambda i: (i, 0)),
        out_shape=jax.ShapeDtypeStruct((N, D), jnp.float32),
    )(nodes, recv, globals_, epsilon, W1[:D], W1[D:], b1.reshape(1, H),
      W2, b2.reshape(1, D))
    return out


# speedup vs baseline: 4.2074x; 1.1796x over previous
import jax
import jax.numpy as jnp
from jax import lax
from jax.experimental import pallas as pl
from jax.experimental.pallas import tpu as pltpu
from jax.experimental.pallas import tpu_sc as plsc

N, E, D, DE, DG, H = 10000, 160000, 256, 16, 128, 512

NC, NS = 2, 16
NW = NC * NS
EH = E // 2
EP = 81920
DP = D // 2
DH = D // 2

G_PER_W = EP // NW
G_IDX = 128
G_NCH = G_PER_W // G_IDX
TLOAD = 624
TLAST = N - (NS - 1) * TLOAD

S_PER_T = EP // NS
S_IDX = 128
S_NCH = S_PER_T // S_IDX
TRASH = 8
ACC_R = N + TRASH
ZR = 632
ZR_LAST = ACC_R - 15 * ZR
WR = 624
WR_TAIL = N - NS * WR

RB_E = 2000
RB_N = 2000


def _sc_gather_body(idx_hbm, table_hbm, out_hbm, idx_v, buf0, buf1, tbl, sem):
    c = lax.axis_index("c")
    s = lax.axis_index("s")
    w = s * NC + c
    base = w * G_PER_W

    @pl.when(s < NS - 1)
    def _load_main():
        pltpu.sync_copy(table_hbm.at[pl.ds(s * TLOAD, TLOAD), :],
                        tbl.at[pl.ds(s * TLOAD, TLOAD), :])

    @pl.when(s == NS - 1)
    def _load_last():
        pltpu.sync_copy(table_hbm.at[pl.ds((NS - 1) * TLOAD, TLAST), :],
                        tbl.at[pl.ds((NS - 1) * TLOAD, TLAST), :])

    pltpu.sync_copy(idx_hbm.at[w], idx_v)
    plsc.subcore_barrier()

    pltpu.async_copy(tbl.at[idx_v.at[0]], buf0, sem)

    def pair(p, _):
        i0 = 2 * p
        pltpu.make_async_copy(tbl.at[idx_v.at[i0]], buf0, sem).wait()
        pltpu.async_copy(tbl.at[idx_v.at[i0 + 1]], buf1, sem)
        pltpu.sync_copy(buf0, out_hbm.at[pl.ds(base + i0 * G_IDX, G_IDX), :])
        pltpu.make_async_copy(tbl.at[idx_v.at[i0 + 1]], buf1, sem).wait()

        @pl.when(p < G_NCH // 2 - 1)
        def _prefetch():
            pltpu.async_copy(tbl.at[idx_v.at[i0 + 2]], buf0, sem)

        pltpu.sync_copy(buf1, out_hbm.at[pl.ds(base + (i0 + 1) * G_IDX, G_IDX), :])
        return 0

    lax.fori_loop(0, G_NCH // 2, pair, 0)


def _sc_scatter_body(ridx_hbm, e_hbm, out_hbm, idx_v, buf0, buf1, acc, sem):
    c = lax.axis_index("c")
    s = lax.axis_index("s")
    zero16 = jnp.zeros((16,), jnp.float32)

    def zrow(r, _):
        for k in range(DH // 16):
            buf0[r, pl.ds(k * 16, 16)] = zero16
        return 0

    lax.fori_loop(0, S_IDX, zrow, 0)

    @pl.when(s < NS - 1)
    def _zero_main():
        zb = s * ZR
        for t in range(ZR // S_IDX):
            pltpu.sync_copy(buf0, acc.at[pl.ds(zb + t * S_IDX, S_IDX), :])
        zrem = ZR % S_IDX
        pltpu.sync_copy(buf0.at[pl.ds(0, zrem), :],
                        acc.at[pl.ds(zb + ZR - zrem, zrem), :])

    @pl.when(s == NS - 1)
    def _zero_last():
        zb = (NS - 1) * ZR
        for t in range(ZR_LAST // S_IDX):
            pltpu.sync_copy(buf0, acc.at[pl.ds(zb + t * S_IDX, S_IDX), :])
        zrem = ZR_LAST % S_IDX
        pltpu.sync_copy(buf0.at[pl.ds(0, zrem), :],
                        acc.at[pl.ds(zb + ZR_LAST - zrem, zrem), :])

    pltpu.sync_copy(ridx_hbm.at[s], idx_v)
    plsc.subcore_barrier()

    row0 = s * S_PER_T
    col = c * DH
    pltpu.async_copy(e_hbm.at[pl.ds(row0, S_IDX), pl.ds(col, DH)], buf0, sem)

    def pair(p, _):
        i0 = 2 * p
        pltpu.make_async_copy(e_hbm.at[pl.ds(row0 + i0 * S_IDX, S_IDX),
                                       pl.ds(col, DH)], buf0, sem).wait()
        pltpu.async_copy(e_hbm.at[pl.ds(row0 + (i0 + 1) * S_IDX, S_IDX),
                                  pl.ds(col, DH)], buf1, sem)
        pltpu.sync_copy(buf0, acc.at[idx_v.at[i0]], add=True)
        pltpu.make_async_copy(e_hbm.at[pl.ds(row0 + (i0 + 1) * S_IDX, S_IDX),
                                       pl.ds(col, DH)], buf1, sem).wait()

        @pl.when(p < S_NCH // 2 - 1)
        def _prefetch():
            pltpu.async_copy(e_hbm.at[pl.ds(row0 + (i0 + 2) * S_IDX, S_IDX),
                                      pl.ds(col, DH)], buf0, sem)

        pltpu.sync_copy(buf1, acc.at[idx_v.at[i0 + 1]], add=True)
        return 0

    lax.fori_loop(0, S_NCH // 2, pair, 0)
    plsc.subcore_barrier()

    pltpu.sync_copy(acc.at[pl.ds(s * WR, WR), :],
                    out_hbm.at[pl.ds(s * WR, WR), pl.ds(col, DH)])

    @pl.when(s == NS - 1)
    def _write_tail():
        pltpu.sync_copy(acc.at[pl.ds(NS * WR, WR_TAIL), :],
                        out_hbm.at[pl.ds(NS * WR, WR_TAIL), pl.ds(col, DH)])


_gather_call = pl.kernel(
    _sc_gather_body,
    out_type=jax.ShapeDtypeStruct((EP, DP), jnp.int32),
    mesh=plsc.VectorSubcoreMesh(core_axis_name="c", subcore_axis_name="s"),
    scratch_types=[
        pltpu.VMEM((G_NCH, G_IDX), jnp.int32),
        pltpu.VMEM((G_IDX, DP), jnp.int32),
        pltpu.VMEM((G_IDX, DP), jnp.int32),
        pltpu.VMEM_SHARED((N, DP), jnp.int32),
        pltpu.SemaphoreType.DMA,
    ],
)

_scatter_call = pl.kernel(
    _sc_scatter_body,
    out_type=jax.ShapeDtypeStruct((N, D), jnp.float32),
    mesh=plsc.VectorSubcoreMesh(core_axis_name="c", subcore_axis_name="s"),
    scratch_types=[
        pltpu.VMEM((S_NCH, S_IDX), jnp.int32),
        pltpu.VMEM((S_IDX, DH), jnp.float32),
        pltpu.VMEM((S_IDX, DH), jnp.float32),
        pltpu.VMEM_SHARED((ACC_R, DH), jnp.float32),
        pltpu.SemaphoreType.DMA,
    ],
)


def _edge_tc(sent_ref, edges_ref, we_ref, be_ref, out_ref):
    z = jnp.dot(edges_ref[...], we_ref[...], preferred_element_type=jnp.float32)
    packed = sent_ref[...]
    lo = jax.lax.bitcast_convert_type(packed << 16, jnp.float32)
    hi = jax.lax.bitcast_convert_type(packed & jnp.int32(-65536), jnp.float32)
    sent = jnp.concatenate([lo, hi], axis=1)
    x = sent + z + be_ref[...]
    u = jnp.exp(jnp.minimum(x, 30.0))
    u = u * (u + 2.0)
    out_ref[...] = x * u / (u + 2.0)


def _mlp_tc(nodes_ref, ra_ref, rb_ref, g_ref, eps_ref, w1a_ref, w1b_ref,
            b1_ref, w2_ref, b2_ref, out_ref):
    h = ((1.0 + eps_ref[...]) * nodes_ref[...] + ra_ref[...] + rb_ref[...])
    gv = jnp.dot(g_ref[...], w1b_ref[...], preferred_element_type=jnp.float32) + b1_ref[...]
    t = jnp.maximum(jnp.dot(h, w1a_ref[...], preferred_element_type=jnp.float32) + gv, 0.0)
    out_ref[...] = jnp.dot(t, w2_ref[...], preferred_element_type=jnp.float32) + b2_ref[...]


def _edge_call(sent, edges_h, W_e_kernel, be_row):
    return pl.pallas_call(
        _edge_tc,
        grid=(EH // RB_E,),
        in_specs=[
            pl.BlockSpec((RB_E, DP), lambda i: (i, 0)),
            pl.BlockSpec((RB_E, DE), lambda i: (i, 0)),
            pl.BlockSpec((DE, D), lambda i: (0, 0)),
            pl.BlockSpec((1, D), lambda i: (0, 0)),
        ],
        out_specs=pl.BlockSpec((RB_E, D), lambda i: (i, 0)),
        out_shape=jax.ShapeDtypeStruct((EP, D), jnp.float32),
    )(sent, edges_h, W_e_kernel, be_row)


def kernel(nodes, edges, globals_, senders, receivers, epsilon,
           W_e_kernel, W_e_bias, W1, b1, W2, b2):
    lo16 = jax.lax.bitcast_convert_type(
        nodes[:, :DH].astype(jnp.bfloat16), jnp.uint16).astype(jnp.uint32)
    hi16 = jax.lax.bitcast_convert_type(
        nodes[:, DH:].astype(jnp.bfloat16), jnp.uint16).astype(jnp.uint32)
    nodes_packed = ((hi16 << 16) | lo16).astype(jnp.int32)

    idx_pad = jnp.zeros((EP - EH,), jnp.int32)
    trash_pad = N + (jnp.arange(EP - EH, dtype=jnp.int32) % TRASH)
    be_row = W_e_bias.reshape(1, D)

    sent_a = _gather_call(
        jnp.concatenate([senders[:EH], idx_pad]).reshape(NW, G_NCH, G_IDX),
        nodes_packed)
    sent_b = _gather_call(
        jnp.concatenate([senders[EH:], idx_pad]).reshape(NW, G_NCH, G_IDX),
        nodes_packed)

    e_a = _edge_call(sent_a, edges[:EH], W_e_kernel, be_row)
    e_b = _edge_call(sent_b, edges[EH:], W_e_kernel, be_row)

    recv_a = _scatter_call(
        jnp.concatenate([receivers[:EH], trash_pad]).reshape(NS, S_NCH, S_IDX),
        e_a)
    recv_b = _scatter_call(
        jnp.concatenate([receivers[EH:], trash_pad]).reshape(NS, S_NCH, S_IDX),
        e_b)

    out = pl.pallas_call(
        _mlp_tc,
        grid=(N // RB_N,),
        in_specs=[
            pl.BlockSpec((RB_N, D), lambda i: (i, 0)),
            pl.BlockSpec((RB_N, D), lambda i: (i, 0)),
            pl.BlockSpec((RB_N, D), lambda i: (i, 0)),
            pl.BlockSpec((1, DG), lambda i: (0, 0)),
            pl.BlockSpec((1, 1), lambda i: (0, 0)),
            pl.BlockSpec((D, H), lambda i: (0, 0)),
            pl.BlockSpec((DG, H), lambda i: (0, 0)),
            pl.BlockSpec((1, H), lambda i: (0, 0)),
            pl.BlockSpec((H, D), lambda i: (0, 0)),
            pl.BlockSpec((1, D), lambda i: (0, 0)),
        ],
        out_specs=pl.BlockSpec((RB_N, D), lambda i: (i, 0)),
        out_shape=jax.ShapeDtypeStruct((N, D), jnp.float32),
    )(nodes, recv_a, recv_b, globals_, epsilon, W1[:D], W1[D:],
      b1.reshape(1, H), W2, b2.reshape(1, D))
    return out


# speedup vs baseline: 4.3460x; 1.0329x over previous
import jax
import jax.numpy as jnp
from jax import lax
from jax.experimental import pallas as pl
from jax.experimental.pallas import tpu as pltpu
from jax.experimental.pallas import tpu_sc as plsc

N, E, D, DE, DG, H = 10000, 160000, 256, 16, 128, 512

NC, NS = 2, 16
NW = NC * NS
EHA = 81920
EHB = E - EHA
EP = 81920
DP = D // 2
DH = D // 2

G_PER_W = EP // NW
G_IDX = 128
G_NCH = G_PER_W // G_IDX
TLOAD = 624
TLAST = N - (NS - 1) * TLOAD

S_PER_T = EP // NS
S_IDX = 128
S_NCH = S_PER_T // S_IDX
TRASH = 8
ACC_R = N + TRASH
ZR = 632
ZR_LAST = ACC_R - 15 * ZR
WR = 624
WR_TAIL = N - NS * WR

RB_EA = 4096
RB_EB = 4880
RB_N = 2000


def _sc_gather_body(idx_hbm, table_hbm, out_hbm, idx_v, buf0, buf1, tbl, sem):
    c = lax.axis_index("c")
    s = lax.axis_index("s")
    w = s * NC + c
    base = w * G_PER_W

    @pl.when(s < NS - 1)
    def _load_main():
        pltpu.sync_copy(table_hbm.at[pl.ds(s * TLOAD, TLOAD), :],
                        tbl.at[pl.ds(s * TLOAD, TLOAD), :])

    @pl.when(s == NS - 1)
    def _load_last():
        pltpu.sync_copy(table_hbm.at[pl.ds((NS - 1) * TLOAD, TLAST), :],
                        tbl.at[pl.ds((NS - 1) * TLOAD, TLAST), :])

    pltpu.sync_copy(idx_hbm.at[w], idx_v)
    plsc.subcore_barrier()

    pltpu.async_copy(tbl.at[idx_v.at[0]], buf0, sem)

    def pair(p, _):
        i0 = 2 * p
        pltpu.make_async_copy(tbl.at[idx_v.at[i0]], buf0, sem).wait()
        pltpu.async_copy(tbl.at[idx_v.at[i0 + 1]], buf1, sem)
        pltpu.sync_copy(buf0, out_hbm.at[pl.ds(base + i0 * G_IDX, G_IDX), :])
        pltpu.make_async_copy(tbl.at[idx_v.at[i0 + 1]], buf1, sem).wait()

        @pl.when(p < G_NCH // 2 - 1)
        def _prefetch():
            pltpu.async_copy(tbl.at[idx_v.at[i0 + 2]], buf0, sem)

        pltpu.sync_copy(buf1, out_hbm.at[pl.ds(base + (i0 + 1) * G_IDX, G_IDX), :])
        return 0

    lax.fori_loop(0, G_NCH // 2, pair, 0)


def _sc_scatter_body(ridx_hbm, e_hbm, out_hbm, idx_v, buf0, buf1, acc, sem):
    c = lax.axis_index("c")
    s = lax.axis_index("s")
    zero16 = jnp.zeros((16,), jnp.float32)

    def zrow(r, _):
        for k in range(DH // 16):
            buf0[r, pl.ds(k * 16, 16)] = zero16
        return 0

    lax.fori_loop(0, S_IDX, zrow, 0)

    @pl.when(s < NS - 1)
    def _zero_main():
        zb = s * ZR
        for t in range(ZR // S_IDX):
            pltpu.sync_copy(buf0, acc.at[pl.ds(zb + t * S_IDX, S_IDX), :])
        zrem = ZR % S_IDX
        pltpu.sync_copy(buf0.at[pl.ds(0, zrem), :],
                        acc.at[pl.ds(zb + ZR - zrem, zrem), :])

    @pl.when(s == NS - 1)
    def _zero_last():
        zb = (NS - 1) * ZR
        for t in range(ZR_LAST // S_IDX):
            pltpu.sync_copy(buf0, acc.at[pl.ds(zb + t * S_IDX, S_IDX), :])
        zrem = ZR_LAST % S_IDX
        pltpu.sync_copy(buf0.at[pl.ds(0, zrem), :],
                        acc.at[pl.ds(zb + ZR_LAST - zrem, zrem), :])

    pltpu.sync_copy(ridx_hbm.at[s], idx_v)
    plsc.subcore_barrier()

    row0 = s * S_PER_T
    col = c * DH
    pltpu.async_copy(e_hbm.at[pl.ds(row0, S_IDX), pl.ds(col, DH)], buf0, sem)

    def pair(p, _):
        i0 = 2 * p
        pltpu.make_async_copy(e_hbm.at[pl.ds(row0 + i0 * S_IDX, S_IDX),
                                       pl.ds(col, DH)], buf0, sem).wait()
        pltpu.async_copy(e_hbm.at[pl.ds(row0 + (i0 + 1) * S_IDX, S_IDX),
                                  pl.ds(col, DH)], buf1, sem)
        pltpu.sync_copy(buf0, acc.at[idx_v.at[i0]], add=True)
        pltpu.make_async_copy(e_hbm.at[pl.ds(row0 + (i0 + 1) * S_IDX, S_IDX),
                                       pl.ds(col, DH)], buf1, sem).wait()

        @pl.when(p < S_NCH // 2 - 1)
        def _prefetch():
            pltpu.async_copy(e_hbm.at[pl.ds(row0 + (i0 + 2) * S_IDX, S_IDX),
                                      pl.ds(col, DH)], buf0, sem)

        pltpu.sync_copy(buf1, acc.at[idx_v.at[i0 + 1]], add=True)
        return 0

    lax.fori_loop(0, S_NCH // 2, pair, 0)
    plsc.subcore_barrier()

    pltpu.sync_copy(acc.at[pl.ds(s * WR, WR), :],
                    out_hbm.at[pl.ds(s * WR, WR), pl.ds(col, DH)])

    @pl.when(s == NS - 1)
    def _write_tail():
        pltpu.sync_copy(acc.at[pl.ds(NS * WR, WR_TAIL), :],
                        out_hbm.at[pl.ds(NS * WR, WR_TAIL), pl.ds(col, DH)])


_gather_call = pl.kernel(
    _sc_gather_body,
    out_type=jax.ShapeDtypeStruct((EP, DP), jnp.int32),
    mesh=plsc.VectorSubcoreMesh(core_axis_name="c", subcore_axis_name="s"),
    scratch_types=[
        pltpu.VMEM((G_NCH, G_IDX), jnp.int32),
        pltpu.VMEM((G_IDX, DP), jnp.int32),
        pltpu.VMEM((G_IDX, DP), jnp.int32),
        pltpu.VMEM_SHARED((N, DP), jnp.int32),
        pltpu.SemaphoreType.DMA,
    ],
)

_scatter_call = pl.kernel(
    _sc_scatter_body,
    out_type=jax.ShapeDtypeStruct((N, D), jnp.float32),
    mesh=plsc.VectorSubcoreMesh(core_axis_name="c", subcore_axis_name="s"),
    scratch_types=[
        pltpu.VMEM((S_NCH, S_IDX), jnp.int32),
        pltpu.VMEM((S_IDX, DH), jnp.float32),
        pltpu.VMEM((S_IDX, DH), jnp.float32),
        pltpu.VMEM_SHARED((ACC_R, DH), jnp.float32),
        pltpu.SemaphoreType.DMA,
    ],
)


def _edge_tc(sent_ref, edges_ref, we_ref, be_ref, out_ref):
    z = jnp.dot(edges_ref[...], we_ref[...], preferred_element_type=jnp.float32)
    packed = sent_ref[...]
    lo = jax.lax.bitcast_convert_type(packed << 16, jnp.float32)
    hi = jax.lax.bitcast_convert_type(packed & jnp.int32(-65536), jnp.float32)
    sent = jnp.concatenate([lo, hi], axis=1)
    x = sent + z + be_ref[...]
    u = jnp.exp(jnp.minimum(x, 30.0))
    u = u * (u + 2.0)
    out_ref[...] = x * u / (u + 2.0)


def _mlp_tc(nodes_ref, ra_ref, rb_ref, g_ref, eps_ref, w1a_ref, w1b_ref,
            b1_ref, w2_ref, b2_ref, out_ref):
    h = ((1.0 + eps_ref[...]) * nodes_ref[...] + ra_ref[...] + rb_ref[...])
    gv = jnp.dot(g_ref[...], w1b_ref[...], preferred_element_type=jnp.float32) + b1_ref[...]
    t = jnp.maximum(jnp.dot(h, w1a_ref[...], preferred_element_type=jnp.float32) + gv, 0.0)
    out_ref[...] = jnp.dot(t, w2_ref[...], preferred_element_type=jnp.float32) + b2_ref[...]


def _edge_call(sent, edges_h, W_e_kernel, be_row, rb, n_real):
    return pl.pallas_call(
        _edge_tc,
        grid=(n_real // rb,),
        in_specs=[
            pl.BlockSpec((rb, DP), lambda i: (i, 0)),
            pl.BlockSpec((rb, DE), lambda i: (i, 0)),
            pl.BlockSpec((DE, D), lambda i: (0, 0)),
            pl.BlockSpec((1, D), lambda i: (0, 0)),
        ],
        out_specs=pl.BlockSpec((rb, D), lambda i: (i, 0)),
        out_shape=jax.ShapeDtypeStruct((EP, D), jnp.float32),
    )(sent, edges_h, W_e_kernel, be_row)


def kernel(nodes, edges, globals_, senders, receivers, epsilon,
           W_e_kernel, W_e_bias, W1, b1, W2, b2):
    lo16 = jax.lax.bitcast_convert_type(
        nodes[:, :DH].astype(jnp.bfloat16), jnp.uint16).astype(jnp.uint32)
    hi16 = jax.lax.bitcast_convert_type(
        nodes[:, DH:].astype(jnp.bfloat16), jnp.uint16).astype(jnp.uint32)
    nodes_packed = ((hi16 << 16) | lo16).astype(jnp.int32)

    idx_pad = jnp.zeros((EP - EHB,), jnp.int32)
    trash_pad = N + (jnp.arange(EP - EHB, dtype=jnp.int32) % TRASH)
    be_row = W_e_bias.reshape(1, D)

    sent_a = _gather_call(
        senders[:EHA].reshape(NW, G_NCH, G_IDX), nodes_packed)
    sent_b = _gather_call(
        jnp.concatenate([senders[EHA:], idx_pad]).reshape(NW, G_NCH, G_IDX),
        nodes_packed)

    e_a = _edge_call(sent_a, edges[:EHA], W_e_kernel, be_row, RB_EA, EHA)
    e_b = _edge_call(sent_b, edges[EHA:], W_e_kernel, be_row, RB_EB, EHB)

    recv_a = _scatter_call(
        receivers[:EHA].reshape(NS, S_NCH, S_IDX), e_a)
    recv_b = _scatter_call(
        jnp.concatenate([receivers[EHA:], trash_pad]).reshape(NS, S_NCH, S_IDX),
        e_b)

    out = pl.pallas_call(
        _mlp_tc,
        grid=(N // RB_N,),
        in_specs=[
            pl.BlockSpec((RB_N, D), lambda i: (i, 0)),
            pl.BlockSpec((RB_N, D), lambda i: (i, 0)),
            pl.BlockSpec((RB_N, D), lambda i: (i, 0)),
            pl.BlockSpec((1, DG), lambda i: (0, 0)),
            pl.BlockSpec((1, 1), lambda i: (0, 0)),
            pl.BlockSpec((D, H), lambda i: (0, 0)),
            pl.BlockSpec((DG, H), lambda i: (0, 0)),
            pl.BlockSpec((1, H), lambda i: (0, 0)),
            pl.BlockSpec((H, D), lambda i: (0, 0)),
            pl.BlockSpec((1, D), lambda i: (0, 0)),
        ],
        out_specs=pl.BlockSpec((RB_N, D), lambda i: (i, 0)),
        out_shape=jax.ShapeDtypeStruct((N, D), jnp.float32),
    )(nodes, recv_a, recv_b, globals_, epsilon, W1[:D], W1[D:],
      b1.reshape(1, H), W2, b2.reshape(1, D))
    return out
